# baseline (speedup 1.0000x reference)
import jax
import jax.numpy as jnp
from jax.experimental import pallas as pl
from jax.experimental.pallas import tpu as pltpu

E_B = 0.05
E_N = 0.006
D_DECAY = 0.995
INPUT_DIM = 150528
BATCH = 64
M = BATCH + 2

_NC = 4
_KC = INPUT_DIM // _NC


_DN_T = (((1,), (1,)), ((), ()))
_DN = (((1,), (0,)), ((), ()))


def _dot(a, b, dn):
    return jax.lax.dot_general(a, b, dn, preferred_element_type=jnp.float32)


def _fused_kernel(n_ref, x_ref, out_ref, err_ref,
                  hi_ref, g_ref, c_ref):
    ph = pl.program_id(0)
    j = pl.program_id(1)

    @pl.when(ph == 0)
    def _():
        x = x_ref[...]
        n = n_ref[...]
        hi_x = x.astype(jnp.bfloat16)
        hi_n = n.astype(jnp.bfloat16)
        hi_ref[j] = hi_x
        hix32 = hi_x.astype(jnp.float32)
        hin32 = hi_n.astype(jnp.float32)
        hi = jnp.concatenate([hi_x, hi_n], axis=0)
        lo = jnp.concatenate(
            [(x - hix32).astype(jnp.bfloat16),
             (n - hin32).astype(jnp.bfloat16)], axis=0)
        d1 = _dot(hi, hi, _DN_T)
        d2 = _dot(hi, lo, _DN_T)
        g = d1 + d2 + d2.T

        @pl.when(j == 0)
        def _():
            g_ref[...] = g

        @pl.when(j != 0)
        def _():
            g_ref[...] += g

    @pl.when((ph == 1) & (j == 0))
    def _():
        lane = jax.lax.broadcasted_iota(jnp.int32, (1, M), 1)
        f32 = jnp.float32
        c0 = (lane == BATCH).astype(f32)
        c1 = (lane == BATCH + 1).astype(f32)
        cg0 = g_ref[BATCH:BATCH + 1, :]
        cg1 = g_ref[BATCH + 1:BATCH + 2, :]
        err = jnp.zeros((1, M), f32)

        def body(p, carry):
            c0, c1, cg0, cg1, err = carry
            onehot = (lane == p).astype(f32)
            gp = g_ref[pl.ds(p, 1), :]
            gpp = jnp.sum(gp * onehot)
            d0 = jnp.sum(cg0 * c0) - 2.0 * jnp.sum(cg0 * onehot) + gpp
            d1 = jnp.sum(cg1 * c1) - 2.0 * jnp.sum(cg1 * onehot) + gpp
            is0 = d0 <= d1
            cb = jnp.where(is0, c0, c1)
            cgb = jnp.where(is0, cg0, cg1)
            cs = jnp.where(is0, c1, c0)
            cgs = jnp.where(is0, cg1, cg0)
            cb_new = (1.0 - E_B) * cb + E_B * onehot
            cgb_new = (1.0 - E_B) * cgb + E_B * gp
            cs_new = cs + E_N * (onehot - cb)
            cgs_new = cgs + E_N * (gp - cgb)
            c0n = jnp.where(is0, cb_new, cs_new)
            c1n = jnp.where(is0, cs_new, cb_new)
            cg0n = jnp.where(is0, cgb_new, cgs_new)
            cg1n = jnp.where(is0, cgs_new, cgb_new)
            db = jnp.where(is0, d0, d1)
            bmask = jnp.where(is0, (lane == 0).astype(f32),
                              (lane == 1).astype(f32))
            err = (err + db * bmask) * D_DECAY
            return c0n, c1n, cg0n, cg1n, err

        c0, c1, cg0, cg1, err = jax.lax.fori_loop(
            0, BATCH, body, (c0, c1, cg0, cg1, err))
        err_ref[...] = err
        c_ref[...] = jnp.concatenate(
            [c0, c1, jnp.zeros((6, M), jnp.float32)], axis=0)

    @pl.when(ph == 1)
    def _():
        cm = c_ref[...]
        hi_c = cm.astype(jnp.bfloat16)
        lo_c = (cm - hi_c.astype(jnp.float32)).astype(jnp.bfloat16)
        hi_x = hi_ref[j]
        n = n_ref[...]
        hi_n = n.astype(jnp.bfloat16)
        lo_n = (n - hi_n.astype(jnp.float32)).astype(jnp.bfloat16)
        out8 = (_dot(hi_c[:, 0:BATCH], hi_x, _DN)
                + _dot(lo_c[:, 0:BATCH], hi_x, _DN)
                + _dot(hi_c[:, BATCH:M], hi_n, _DN)
                + _dot(hi_c[:, BATCH:M], lo_n, _DN)
                + _dot(lo_c[:, BATCH:M], hi_n, _DN))
        out_ref[...] = out8[0:2, :]


def kernel(images, labels, nodes, local_error, edges):
    del labels
    nodes_out, err_row = pl.pallas_call(
        _fused_kernel,
        grid=(2, _NC),
        in_specs=[
            pl.BlockSpec((2, _KC), lambda p, j: (0, j)),
            pl.BlockSpec((BATCH, _KC),
                         lambda p, j: (0, j * (1 - p) + (_NC - 1) * p)),
        ],
        out_specs=[
            pl.BlockSpec((2, _KC), lambda p, j: (0, j * p)),
            pl.BlockSpec((1, M), lambda p, j: (0, 0)),
        ],
        out_shape=[
            jax.ShapeDtypeStruct((2, INPUT_DIM), jnp.float32),
            jax.ShapeDtypeStruct((1, M), jnp.float32),
        ],
        scratch_shapes=[
            pltpu.VMEM((_NC, BATCH, _KC), jnp.bfloat16),
            pltpu.VMEM((M, M), jnp.float32),
            pltpu.VMEM((8, M), jnp.float32),
        ],
    )(nodes, images)

    local_error_out = err_row[0, 0:2] + local_error * (D_DECAY ** BATCH)
    return nodes_out, local_error_out, edges
